# baseline (speedup 1.0000x reference)
import functools

import jax
import jax.numpy as jnp
from jax import lax
from jax.experimental import pallas as pl
from jax.experimental.pallas import tpu as pltpu
from jax.experimental.pallas import tpu_sc as plsc

B = 16384
D = 64
NUM_TABLES = 4
_NC, _NS = 2, 16
_NW = _NC * _NS
_BPW = B // _NW
_CH = 128
_NCHUNK = _BPW // _CH
_NSTEP = NUM_TABLES * _NCHUNK


def _gather_body(type_hbm, loc_hbm, src_hbm, tgt_hbm,
                 tab_type, tab_loc, tab_src, tab_tgt,
                 out_hbm, idx_v, pair_v, buf0, buf1, gsem, wsem):
    wid = lax.axis_index("s") * _NC + lax.axis_index("c")
    base = wid * _BPW
    idx_refs = (type_hbm, loc_hbm, src_hbm, tgt_hbm)
    tabs = (tab_type, tab_loc, tab_src, tab_tgt)
    bufs = (buf0, buf1)

    for t in range(NUM_TABLES):
        for c in range(_NCHUNK):
            pltpu.sync_copy(idx_refs[t].at[pl.ds(base + c * _CH, _CH)],
                            idx_v.at[t * _NCHUNK + c])
    for k in range(_NSTEP):
        for v in range(_CH // 16):
            pair_v[k, pl.ds(v * 16, 16)] = jax.lax.shift_right_logical(
                idx_v[k, pl.ds(v * 16, 16)], 1)

    def start_gather(k):
        t = k // _NCHUNK
        return pltpu.async_copy(tabs[t].at[pair_v.at[k]], bufs[k % 2], gsem)

    def start_write(k):
        t, c = divmod(k, _NCHUNK)
        return pltpu.async_copy(
            bufs[k % 2], out_hbm.at[t, pl.ds(base + c * _CH, _CH)], wsem)

    writes = [None] * _NSTEP
    g = start_gather(0)
    for k in range(_NSTEP):
        g_next = None
        if k + 1 < _NSTEP:
            if k - 1 >= 0:
                writes[k - 1].wait()
            g_next = start_gather(k + 1)
        g.wait()
        writes[k] = start_write(k)
        g = g_next
    writes[_NSTEP - 2].wait()
    writes[_NSTEP - 1].wait()


@functools.cache
def _gather():
    return functools.partial(
        pl.kernel,
        mesh=plsc.VectorSubcoreMesh(core_axis_name="c", subcore_axis_name="s"),
        out_type=jax.ShapeDtypeStruct((NUM_TABLES, B, 2 * D), jnp.float32),
        scratch_types=[
            pltpu.VMEM((_NSTEP, _CH), jnp.int32),
            pltpu.VMEM((_NSTEP, _CH), jnp.int32),
            pltpu.VMEM((_CH, 2 * D), jnp.float32),
            pltpu.VMEM((_CH, 2 * D), jnp.float32),
            pltpu.SemaphoreType.DMA,
            pltpu.SemaphoreType.DMA,
        ],
    )(_gather_body)


def _mlp_body(x_ref, idx_ref, ln1g_ref, ln1b_ref, w1_ref, b1_ref, w2_ref,
              b2_ref, ln2g_ref, ln2b_ref, out_ref):
    parts = []
    for t in range(NUM_TABLES):
        xt = x_ref[t]
        m = (idx_ref[:, t:t + 1] & 1).astype(jnp.float32)
        left = xt[:, :D]
        right = xt[:, D:]
        parts.append(left + (right - left) * m)
    xc = jnp.concatenate(parts, axis=-1)
    mu = jnp.mean(xc, axis=-1, keepdims=True)
    xm = xc - mu
    var = jnp.mean(xm * xm, axis=-1, keepdims=True)
    h = xm * lax.rsqrt(var + 1e-5) * ln1g_ref[...] + ln1b_ref[...]
    h = jnp.dot(h, w1_ref[...], preferred_element_type=jnp.float32)
    h = jnp.maximum(h + b1_ref[...], 0.0)
    h = jnp.dot(h, w2_ref[...], preferred_element_type=jnp.float32)
    h = h + b2_ref[...]
    mu2 = jnp.mean(h, axis=-1, keepdims=True)
    hm = h - mu2
    var2 = jnp.mean(hm * hm, axis=-1, keepdims=True)
    out_ref[...] = hm * lax.rsqrt(var2 + 1e-5) * ln2g_ref[...] + ln2b_ref[...]


_R = 1024


def _mlp(x, idx, ln1_g, ln1_b, W1, b1, W2, b2, ln2_g, ln2_b):
    grid = (B // _R,)
    full = lambda shape: pl.BlockSpec(shape, lambda i: (0, 0))
    return pl.pallas_call(
        _mlp_body,
        grid=grid,
        in_specs=[
            pl.BlockSpec((NUM_TABLES, _R, 2 * D), lambda i: (0, i, 0)),
            pl.BlockSpec((_R, NUM_TABLES), lambda i: (i, 0)),
            full((1, 4 * D)), full((1, 4 * D)),
            full((4 * D, 2 * D)), full((1, 2 * D)),
            full((2 * D, D)), full((1, D)),
            full((1, D)), full((1, D)),
        ],
        out_specs=pl.BlockSpec((_R, D), lambda i: (i, 0)),
        out_shape=jax.ShapeDtypeStruct((B, D), jnp.float32),
    )(x, idx, ln1_g.reshape(1, -1), ln1_b.reshape(1, -1), W1,
      b1.reshape(1, -1), W2, b2.reshape(1, -1), ln2_g.reshape(1, -1),
      ln2_b.reshape(1, -1))


def kernel(type_idx, loc_idx, src_idx, tgt_idx, emb_type, emb_loc,
           source_emb, target_emb, ln1_g, ln1_b, W1, b1, W2, b2,
           ln2_g, ln2_b):
    idxs = [type_idx.astype(jnp.int32), loc_idx.astype(jnp.int32),
            src_idx.astype(jnp.int32), tgt_idx.astype(jnp.int32)]
    pairs = _gather()(
        *idxs,
        emb_type.reshape(-1, 2 * D), emb_loc.reshape(-1, 2 * D),
        source_emb.reshape(-1, 2 * D), target_emb.reshape(-1, 2 * D))
    idx_mat = jnp.stack(idxs, axis=1)
    return _mlp(pairs, idx_mat, ln1_g, ln1_b, W1, b1, W2, b2, ln2_g, ln2_b)


# speedup vs baseline: 1.0063x; 1.0063x over previous
import functools

import jax
import jax.numpy as jnp
from jax import lax
from jax.experimental import pallas as pl
from jax.experimental.pallas import tpu as pltpu
from jax.experimental.pallas import tpu_sc as plsc

B = 16384
D = 64
_NC, _NS = 2, 16
_NW = _NC * _NS
_BPW = B // _NW
_CH = 128
_NCHUNK = _BPW // _CH


def _make_gather_body(num_tables):
    nstep = num_tables * _NCHUNK

    def body(*refs):
        idx_refs = refs[:num_tables]
        tabs = refs[num_tables:2 * num_tables]
        out_hbm = refs[2 * num_tables]
        idx_v, pair_v, buf0, buf1, gsem, wsem = refs[2 * num_tables + 1:]
        bufs = (buf0, buf1)
        wid = lax.axis_index("s") * _NC + lax.axis_index("c")
        base = wid * _BPW

        for t in range(num_tables):
            for c in range(_NCHUNK):
                pltpu.sync_copy(idx_refs[t].at[pl.ds(base + c * _CH, _CH)],
                                idx_v.at[t * _NCHUNK + c])
        for k in range(nstep):
            for v in range(_CH // 16):
                pair_v[k, pl.ds(v * 16, 16)] = jax.lax.shift_right_logical(
                    idx_v[k, pl.ds(v * 16, 16)], 1)

        def start_gather(k):
            t = k // _NCHUNK
            return pltpu.async_copy(tabs[t].at[pair_v.at[k]], bufs[k % 2],
                                    gsem)

        def start_write(k):
            t, c = divmod(k, _NCHUNK)
            return pltpu.async_copy(
                bufs[k % 2], out_hbm.at[t, pl.ds(base + c * _CH, _CH)], wsem)

        writes = [None] * nstep
        g = start_gather(0)
        for k in range(nstep):
            g_next = None
            if k + 1 < nstep:
                if k - 1 >= 0:
                    writes[k - 1].wait()
                g_next = start_gather(k + 1)
            g.wait()
            writes[k] = start_write(k)
            g = g_next
        if nstep >= 2:
            writes[nstep - 2].wait()
        writes[nstep - 1].wait()

    return body


@functools.cache
def _gather(num_tables):
    return functools.partial(
        pl.kernel,
        mesh=plsc.VectorSubcoreMesh(core_axis_name="c", subcore_axis_name="s"),
        out_type=jax.ShapeDtypeStruct((num_tables, B, 2 * D), jnp.float32),
        scratch_types=[
            pltpu.VMEM((num_tables * _NCHUNK, _CH), jnp.int32),
            pltpu.VMEM((num_tables * _NCHUNK, _CH), jnp.int32),
            pltpu.VMEM((_CH, 2 * D), jnp.float32),
            pltpu.VMEM((_CH, 2 * D), jnp.float32),
            pltpu.SemaphoreType.DMA,
            pltpu.SemaphoreType.DMA,
        ],
    )(_make_gather_body(num_tables))


def _mlp_body(xa_ref, xb_ref, xc_ref, idx_ref, ln1g_ref, ln1b_ref, w1_ref,
              b1_ref, w2_ref, b2_ref, ln2g_ref, ln2b_ref, out_ref):
    planes = [xa_ref[0], xa_ref[1], xb_ref[0], xc_ref[0]]
    parts = []
    for t in range(4):
        xt = planes[t]
        m = (idx_ref[:, t:t + 1] & 1).astype(jnp.float32)
        left = xt[:, :D]
        right = xt[:, D:]
        parts.append(left + (right - left) * m)
    xc = jnp.concatenate(parts, axis=-1)
    mu = jnp.mean(xc, axis=-1, keepdims=True)
    xm = xc - mu
    var = jnp.mean(xm * xm, axis=-1, keepdims=True)
    h = xm * lax.rsqrt(var + 1e-5) * ln1g_ref[...] + ln1b_ref[...]
    h = jnp.dot(h, w1_ref[...], preferred_element_type=jnp.float32)
    h = jnp.maximum(h + b1_ref[...], 0.0)
    h = jnp.dot(h, w2_ref[...], preferred_element_type=jnp.float32)
    h = h + b2_ref[...]
    mu2 = jnp.mean(h, axis=-1, keepdims=True)
    hm = h - mu2
    var2 = jnp.mean(hm * hm, axis=-1, keepdims=True)
    out_ref[...] = hm * lax.rsqrt(var2 + 1e-5) * ln2g_ref[...] + ln2b_ref[...]


_R = 1024


def _mlp(xa, xb, xc, idx, ln1_g, ln1_b, W1, b1, W2, b2, ln2_g, ln2_b):
    grid = (B // _R,)
    full = lambda shape: pl.BlockSpec(shape, lambda i: (0, 0))
    return pl.pallas_call(
        _mlp_body,
        grid=grid,
        in_specs=[
            pl.BlockSpec((2, _R, 2 * D), lambda i: (0, i, 0)),
            pl.BlockSpec((1, _R, 2 * D), lambda i: (0, i, 0)),
            pl.BlockSpec((1, _R, 2 * D), lambda i: (0, i, 0)),
            pl.BlockSpec((_R, 4), lambda i: (i, 0)),
            full((1, 4 * D)), full((1, 4 * D)),
            full((4 * D, 2 * D)), full((1, 2 * D)),
            full((2 * D, D)), full((1, D)),
            full((1, D)), full((1, D)),
        ],
        out_specs=pl.BlockSpec((_R, D), lambda i: (i, 0)),
        out_shape=jax.ShapeDtypeStruct((B, D), jnp.float32),
    )(xa, xb, xc, idx, ln1_g.reshape(1, -1), ln1_b.reshape(1, -1), W1,
      b1.reshape(1, -1), W2, b2.reshape(1, -1), ln2_g.reshape(1, -1),
      ln2_b.reshape(1, -1))


def kernel(type_idx, loc_idx, src_idx, tgt_idx, emb_type, emb_loc,
           source_emb, target_emb, ln1_g, ln1_b, W1, b1, W2, b2,
           ln2_g, ln2_b):
    ti = type_idx.astype(jnp.int32)
    li = loc_idx.astype(jnp.int32)
    si = src_idx.astype(jnp.int32)
    gi = tgt_idx.astype(jnp.int32)
    xa = _gather(2)(ti, li, emb_type.reshape(-1, 2 * D),
                    emb_loc.reshape(-1, 2 * D))
    xb = _gather(1)(si, source_emb.reshape(-1, 2 * D))
    xc = _gather(1)(gi, target_emb.reshape(-1, 2 * D))
    idx_mat = jnp.stack([ti, li, si, gi], axis=1)
    return _mlp(xa, xb, xc, idx_mat, ln1_g, ln1_b, W1, b1, W2, b2,
                ln2_g, ln2_b)
